# baseline (speedup 1.0000x reference)
import functools

import jax
import jax.numpy as jnp
from jax import lax
from jax.experimental import pallas as pl
from jax.experimental.pallas import tpu as pltpu
from jax.experimental.pallas import tpu_sc as plsc

_B, _C, _DP = 4, 2048, 256
_DR = 64
_TOPK = 8
_TQ = 512

_NB = _B * _C * _TOPK
_NW = 32
_BPW = _NB // _NW
_CHUNK = 128
_NCHUNK = _BPW // _CHUNK


def _compute_logits_tile(q_ref, k_ref, wq_ref, wk_ref, xk_ref):
    i = pl.program_id(1)

    @pl.when(i == 0)
    def _():
        xk_ref[...] = jnp.dot(
            k_ref[0], wk_ref[...], preferred_element_type=jnp.float32
        )

    x_q = jnp.dot(q_ref[0], wq_ref[...], preferred_element_type=jnp.float32)
    return lax.dot_general(
        x_q, xk_ref[...], (((1,), (1,)), ((), ())),
        preferred_element_type=jnp.float32,
    )


def _logits_body(q_ref, k_ref, wq_ref, wk_ref, logits_ref, xk_ref):
    logits_ref[0] = _compute_logits_tile(q_ref, k_ref, wq_ref, wk_ref, xk_ref)


def _topk_body(q_ref, k_ref, wq_ref, wk_ref, idx_ref, xk_ref):
    b = pl.program_id(0)
    logits = _compute_logits_tile(q_ref, k_ref, wq_ref, wk_ref, xk_ref)

    half = _C // 2
    colL = lax.broadcasted_iota(jnp.int32, (_TQ, half), 1).astype(jnp.float32)
    colR = colL + float(half)
    lhs = logits[:, :half]
    rhs = logits[:, half:]
    cond = lhs >= rhs
    P = jnp.maximum(lhs, rhs)
    NV = jnp.minimum(lhs, rhs)
    PC = jnp.where(cond, colL, colR)
    NC = jnp.where(cond, colR, colL)
    t_iota = lax.broadcasted_iota(jnp.int32, (_TQ, _TOPK), 1)
    idx_acc = jnp.zeros((_TQ, _TOPK), jnp.int32)
    for t in range(_TOPK):
        m = jnp.max(P, axis=1, keepdims=True)
        amaxc = jnp.min(jnp.where(P == m, PC, float(_C)), axis=1,
                        keepdims=True)
        winner = PC == amaxc
        idx_acc = jnp.where(t_iota == t, amaxc.astype(jnp.int32), idx_acc)
        P = jnp.where(winner, NV, P)
        PC = jnp.where(winner, NC, PC)
        NV = jnp.where(winner, -jnp.inf, NV)
    idx_ref[0] = idx_acc + b * _C


_IN_SPECS = [
    pl.BlockSpec((1, _TQ, _DP), lambda b, i: (b, i, 0)),
    pl.BlockSpec((1, _C, _DP), lambda b, i: (b, 0, 0)),
    pl.BlockSpec((_DP, _DR), lambda b, i: (0, 0)),
    pl.BlockSpec((_DP, _DR), lambda b, i: (0, 0)),
]


def _topk(q, k, Wq, Wk, interpret=False):
    return pl.pallas_call(
        _topk_body,
        grid=(_B, _C // _TQ),
        in_specs=_IN_SPECS,
        out_specs=pl.BlockSpec((1, _TQ, _TOPK), lambda b, i: (b, i, 0)),
        out_shape=jax.ShapeDtypeStruct((_B, _C, _TOPK), jnp.int32),
        scratch_shapes=[pltpu.VMEM((_C, _DR), jnp.float32)],
        interpret=interpret,
    )(q, k, Wq, Wk)


def _logits(q, k, Wq, Wk, interpret=False):
    return pl.pallas_call(
        _logits_body,
        grid=(_B, _C // _TQ),
        in_specs=_IN_SPECS,
        out_specs=pl.BlockSpec((1, _TQ, _C), lambda b, i: (b, i, 0)),
        out_shape=jax.ShapeDtypeStruct((_B, _C, _C), jnp.float32),
        scratch_shapes=[pltpu.VMEM((_C, _DR), jnp.float32)],
        interpret=interpret,
    )(q, k, Wq, Wk)


_NBUF = 3


def _gather_body(v_hbm, idx_hbm, out_hbm, idx_v, buf0, buf1, buf2,
                 gsem0, gsem1, gsem2, wsem0, wsem1, wsem2):
    wid = lax.axis_index("s") * 2 + lax.axis_index("c")
    base = wid * _BPW
    pltpu.sync_copy(idx_hbm.at[pl.ds(base, _BPW)], idx_v)

    bufs = (buf0, buf1, buf2)
    gsems = (gsem0, gsem1, gsem2)
    wsems = (wsem0, wsem1, wsem2)

    def start_gather(g):
        s = g % _NBUF
        return pltpu.async_copy(
            v_hbm.at[idx_v.at[pl.ds(g * _CHUNK, _CHUNK)]], bufs[s], gsems[s]
        )

    def start_write(g):
        s = g % _NBUF
        return pltpu.async_copy(
            bufs[s], out_hbm.at[pl.ds(base + g * _CHUNK, _CHUNK)], wsems[s]
        )

    gh = [None] * _NBUF
    wh = [None] * _NBUF
    for g in range(_NCHUNK):
        s = g % _NBUF
        if wh[s] is not None:
            wh[s].wait()
        gh[s] = start_gather(g)
        if g >= 1:
            sp = (g - 1) % _NBUF
            gh[sp].wait()
            wh[sp] = start_write(g - 1)
    last = (_NCHUNK - 1) % _NBUF
    gh[last].wait()
    wh[last] = start_write(_NCHUNK - 1)
    for s in range(_NBUF):
        if wh[s] is not None:
            wh[s].wait()


@functools.cache
def _gather_rows():
    return functools.partial(
        pl.kernel,
        out_type=jax.ShapeDtypeStruct((_NB, _DP), jnp.float32),
        mesh=plsc.VectorSubcoreMesh(core_axis_name="c", subcore_axis_name="s"),
        scratch_types=(
            [pltpu.VMEM((_BPW,), jnp.int32)]
            + [pltpu.VMEM((_CHUNK, _DP), jnp.float32)] * _NBUF
            + [pltpu.SemaphoreType.DMA] * (2 * _NBUF)
        ),
    )(_gather_body)


def kernel(q, k, v, Wq, Wk):
    idx_global = _topk(q, k, Wq, Wk)
    flat_idx = idx_global.reshape(_NB)
    v_flat = v.reshape(_B * _C, _DP)
    rec_flat = _gather_rows()(v_flat, flat_idx)
    logits = _logits(q, k, Wq, Wk)
    rec_x = rec_flat.reshape(_B, _C, _TOPK, _DP)
    return (q, rec_x, logits)


# speedup vs baseline: 1.2068x; 1.2068x over previous
import functools

import jax
import jax.numpy as jnp
from jax import lax
from jax.experimental import pallas as pl
from jax.experimental.pallas import tpu as pltpu
from jax.experimental.pallas import tpu_sc as plsc

_B, _C, _DP = 4, 2048, 256
_DR = 64
_TOPK = 8
_TQ = 512
_BH = _B // 2

_NB = _B * _C * _TOPK
_NBH = _NB // 2
_NW = 32
_BPW = _NBH // _NW
_CHUNK = 128
_NCHUNK = _BPW // _CHUNK


def _fused_body(b_base, q_ref, k_ref, wq_ref, wk_ref, logits_ref, idx_ref,
                xk_ref):
    b = pl.program_id(0)
    i = pl.program_id(1)

    @pl.when(i == 0)
    def _():
        xk_ref[...] = jnp.dot(
            k_ref[0], wk_ref[...], preferred_element_type=jnp.float32
        )

    x_q = jnp.dot(q_ref[0], wq_ref[...], preferred_element_type=jnp.float32)
    logits = lax.dot_general(
        x_q, xk_ref[...], (((1,), (1,)), ((), ())),
        preferred_element_type=jnp.float32,
    )
    logits_ref[0] = logits

    half = _C // 2
    colL = lax.broadcasted_iota(jnp.int32, (_TQ, half), 1).astype(jnp.float32)
    colR = colL + float(half)
    lhs = logits[:, :half]
    rhs = logits[:, half:]
    cond = lhs >= rhs
    P = jnp.maximum(lhs, rhs)
    NV = jnp.minimum(lhs, rhs)
    PC = jnp.where(cond, colL, colR)
    NC = jnp.where(cond, colR, colL)
    t_iota = lax.broadcasted_iota(jnp.int32, (_TQ, _TOPK), 1)
    idx_acc = jnp.zeros((_TQ, _TOPK), jnp.int32)
    for t in range(_TOPK):
        m = jnp.max(P, axis=1, keepdims=True)
        amaxc = jnp.min(jnp.where(P == m, PC, float(_C)), axis=1,
                        keepdims=True)
        winner = PC == amaxc
        idx_acc = jnp.where(t_iota == t, amaxc.astype(jnp.int32), idx_acc)
        P = jnp.where(winner, NV, P)
        PC = jnp.where(winner, NC, PC)
        NV = jnp.where(winner, -jnp.inf, NV)
    idx_ref[0] = idx_acc + (b + b_base) * _C


def _half0_call(q, k, Wq, Wk, interpret=False):
    return pl.pallas_call(
        functools.partial(_fused_body, 0),
        grid=(_BH, _C // _TQ),
        in_specs=[
            pl.BlockSpec((1, _TQ, _DP), lambda b, i: (b, i, 0)),
            pl.BlockSpec((1, _C, _DP), lambda b, i: (b, 0, 0)),
            pl.BlockSpec((_DP, _DR), lambda b, i: (0, 0)),
            pl.BlockSpec((_DP, _DR), lambda b, i: (0, 0)),
        ],
        out_specs=[
            pl.BlockSpec((1, _TQ, _C), lambda b, i: (b, i, 0)),
            pl.BlockSpec((1, _TQ, _TOPK), lambda b, i: (b, i, 0)),
        ],
        out_shape=[
            jax.ShapeDtypeStruct((_B, _C, _C), jnp.float32),
            jax.ShapeDtypeStruct((_BH, _C, _TOPK), jnp.int32),
        ],
        interpret=interpret,
        scratch_shapes=[pltpu.VMEM((_C, _DR), jnp.float32)],
    )(q, k, Wq, Wk)


def _half1_call(q, k, Wq, Wk, logits_buf, interpret=False):
    return pl.pallas_call(
        functools.partial(_half1_body, _BH),
        grid=(_BH, _C // _TQ),
        in_specs=[
            pl.BlockSpec((1, _TQ, _DP), lambda b, i: (b + _BH, i, 0)),
            pl.BlockSpec((1, _C, _DP), lambda b, i: (b + _BH, 0, 0)),
            pl.BlockSpec((_DP, _DR), lambda b, i: (0, 0)),
            pl.BlockSpec((_DP, _DR), lambda b, i: (0, 0)),
            pl.BlockSpec(memory_space=pl.ANY),
        ],
        out_specs=[
            pl.BlockSpec((1, _TQ, _C), lambda b, i: (b + _BH, i, 0)),
            pl.BlockSpec((1, _TQ, _TOPK), lambda b, i: (b, i, 0)),
        ],
        out_shape=[
            jax.ShapeDtypeStruct((_B, _C, _C), jnp.float32),
            jax.ShapeDtypeStruct((_BH, _C, _TOPK), jnp.int32),
        ],
        input_output_aliases={4: 0},
        interpret=interpret,
        scratch_shapes=[pltpu.VMEM((_C, _DR), jnp.float32)],
    )(q, k, Wq, Wk, logits_buf)


def _half1_body(b_base, q_ref, k_ref, wq_ref, wk_ref, prev_ref, logits_ref,
                idx_ref, xk_ref):
    del prev_ref
    _fused_body(b_base, q_ref, k_ref, wq_ref, wk_ref, logits_ref, idx_ref,
                xk_ref)


def _gather_half_body(row0, v_hbm, idx_hbm, out_hbm, idx_v, buf0, buf1, buf2,
                      gsem0, gsem1, gsem2, wsem0, wsem1, wsem2):
    wid = lax.axis_index("s") * 2 + lax.axis_index("c")
    base = wid * _BPW
    pltpu.sync_copy(idx_hbm.at[pl.ds(base, _BPW)], idx_v)
    obase = row0 + base

    bufs = (buf0, buf1, buf2)
    gsems = (gsem0, gsem1, gsem2)
    wsems = (wsem0, wsem1, wsem2)

    def start_gather(g):
        s = g % _NBUF
        return pltpu.async_copy(
            v_hbm.at[idx_v.at[pl.ds(g * _CHUNK, _CHUNK)]], bufs[s], gsems[s]
        )

    def start_write(g):
        s = g % _NBUF
        return pltpu.async_copy(
            bufs[s], out_hbm.at[pl.ds(obase + g * _CHUNK, _CHUNK)], wsems[s]
        )

    gh = [None] * _NBUF
    wh = [None] * _NBUF
    for g in range(_NCHUNK):
        s = g % _NBUF
        if wh[s] is not None:
            wh[s].wait()
        gh[s] = start_gather(g)
        if g >= 1:
            sp = (g - 1) % _NBUF
            gh[sp].wait()
            wh[sp] = start_write(g - 1)
    last = (_NCHUNK - 1) % _NBUF
    gh[last].wait()
    wh[last] = start_write(_NCHUNK - 1)
    for s in range(_NBUF):
        if wh[s] is not None:
            wh[s].wait()


_NBUF = 3

_SC_SCRATCH = (
    [pltpu.VMEM((_BPW,), jnp.int32)]
    + [pltpu.VMEM((_CHUNK, _DP), jnp.float32)] * _NBUF
    + [pltpu.SemaphoreType.DMA] * (2 * _NBUF)
)


@functools.cache
def _gather_half0():
    return functools.partial(
        pl.kernel,
        out_type=jax.ShapeDtypeStruct((_NB, _DP), jnp.float32),
        mesh=plsc.VectorSubcoreMesh(core_axis_name="c", subcore_axis_name="s"),
        scratch_types=list(_SC_SCRATCH),
    )(functools.partial(_gather_half_body, 0))


@functools.cache
def _gather_half1():
    def body(v_hbm, idx_hbm, out_ref, *scratch):
        _gather_half_body(_NBH, v_hbm, idx_hbm, out_ref, *scratch)

    return functools.partial(
        pl.kernel,
        out_type=(),
        mesh=plsc.VectorSubcoreMesh(core_axis_name="c", subcore_axis_name="s"),
        scratch_types=list(_SC_SCRATCH),
    )(body)


def kernel(q, k, v, Wq, Wk):
    v_flat = v.reshape(_B * _C, _DP)
    logits_h, idx0 = _half0_call(q, k, Wq, Wk)
    rec = _gather_half0()(v_flat, idx0.reshape(_NBH))
    logits, idx1 = _half1_call(q, k, Wq, Wk, logits_h)
    rec_ref = jax.new_ref(rec)
    _gather_half1()(v_flat, idx1.reshape(_NBH), rec_ref)
    rec_x = rec_ref[...].reshape(_B, _C, _TOPK, _DP)
    return (q, rec_x, logits)


# speedup vs baseline: 1.2257x; 1.0157x over previous
import functools

import jax
import jax.numpy as jnp
from jax import lax
from jax.experimental import pallas as pl
from jax.experimental.pallas import tpu as pltpu
from jax.experimental.pallas import tpu_sc as plsc

_B, _C, _DP = 4, 2048, 256
_DR = 64
_TOPK = 8
_TQ = 1024
_BH = _B // 2

_NB = _B * _C * _TOPK
_NBH = _NB // 2
_NW = 32
_BPW = _NBH // _NW
_CHUNK = 128
_NCHUNK = _BPW // _CHUNK


def _fused_body(b_base, q_ref, k_ref, wq_ref, wk_ref, logits_ref, idx_ref,
                xk_ref):
    b = pl.program_id(0)
    i = pl.program_id(1)

    @pl.when(i == 0)
    def _():
        xk_ref[...] = jnp.dot(
            k_ref[0], wk_ref[...], preferred_element_type=jnp.float32
        )

    x_q = jnp.dot(q_ref[0], wq_ref[...], preferred_element_type=jnp.float32)
    logits = lax.dot_general(
        x_q, xk_ref[...], (((1,), (1,)), ((), ())),
        preferred_element_type=jnp.float32,
    )
    logits_ref[0] = logits

    half = _C // 2
    colL = lax.broadcasted_iota(jnp.int32, (_TQ, half), 1).astype(jnp.float32)
    colR = colL + float(half)
    lhs = logits[:, :half]
    rhs = logits[:, half:]
    cond = lhs >= rhs
    P = jnp.maximum(lhs, rhs)
    NV = jnp.minimum(lhs, rhs)
    PC = jnp.where(cond, colL, colR)
    NC = jnp.where(cond, colR, colL)
    t_iota = lax.broadcasted_iota(jnp.int32, (_TQ, _TOPK), 1)
    idx_acc = jnp.zeros((_TQ, _TOPK), jnp.int32)
    for t in range(_TOPK):
        m = jnp.max(P, axis=1, keepdims=True)
        amaxc = jnp.min(jnp.where(P == m, PC, float(_C)), axis=1,
                        keepdims=True)
        winner = PC == amaxc
        idx_acc = jnp.where(t_iota == t, amaxc.astype(jnp.int32), idx_acc)
        P = jnp.where(winner, NV, P)
        PC = jnp.where(winner, NC, PC)
        NV = jnp.where(winner, -jnp.inf, NV)
    idx_ref[0] = idx_acc + (b + b_base) * _C


def _half0_call(q, k, Wq, Wk, interpret=False):
    return pl.pallas_call(
        functools.partial(_fused_body, 0),
        grid=(_BH, _C // _TQ),
        in_specs=[
            pl.BlockSpec((1, _TQ, _DP), lambda b, i: (b, i, 0)),
            pl.BlockSpec((1, _C, _DP), lambda b, i: (b, 0, 0)),
            pl.BlockSpec((_DP, _DR), lambda b, i: (0, 0)),
            pl.BlockSpec((_DP, _DR), lambda b, i: (0, 0)),
        ],
        out_specs=[
            pl.BlockSpec((1, _TQ, _C), lambda b, i: (b, i, 0)),
            pl.BlockSpec((1, _TQ, _TOPK), lambda b, i: (b, i, 0)),
        ],
        out_shape=[
            jax.ShapeDtypeStruct((_B, _C, _C), jnp.float32),
            jax.ShapeDtypeStruct((_BH, _C, _TOPK), jnp.int32),
        ],
        interpret=interpret,
        scratch_shapes=[pltpu.VMEM((_C, _DR), jnp.float32)],
    )(q, k, Wq, Wk)


def _half1_call(q, k, Wq, Wk, logits_buf, interpret=False):
    return pl.pallas_call(
        functools.partial(_half1_body, _BH),
        grid=(_BH, _C // _TQ),
        in_specs=[
            pl.BlockSpec((1, _TQ, _DP), lambda b, i: (b + _BH, i, 0)),
            pl.BlockSpec((1, _C, _DP), lambda b, i: (b + _BH, 0, 0)),
            pl.BlockSpec((_DP, _DR), lambda b, i: (0, 0)),
            pl.BlockSpec((_DP, _DR), lambda b, i: (0, 0)),
            pl.BlockSpec(memory_space=pl.ANY),
        ],
        out_specs=[
            pl.BlockSpec((1, _TQ, _C), lambda b, i: (b + _BH, i, 0)),
            pl.BlockSpec((1, _TQ, _TOPK), lambda b, i: (b, i, 0)),
        ],
        out_shape=[
            jax.ShapeDtypeStruct((_B, _C, _C), jnp.float32),
            jax.ShapeDtypeStruct((_BH, _C, _TOPK), jnp.int32),
        ],
        input_output_aliases={4: 0},
        interpret=interpret,
        scratch_shapes=[pltpu.VMEM((_C, _DR), jnp.float32)],
    )(q, k, Wq, Wk, logits_buf)


def _half1_body(b_base, q_ref, k_ref, wq_ref, wk_ref, prev_ref, logits_ref,
                idx_ref, xk_ref):
    del prev_ref
    _fused_body(b_base, q_ref, k_ref, wq_ref, wk_ref, logits_ref, idx_ref,
                xk_ref)


def _gather_half_body(row0, v_hbm, idx_hbm, out_hbm, idx_v, buf0, buf1, buf2,
                      gsem0, gsem1, gsem2, wsem0, wsem1, wsem2):
    wid = lax.axis_index("s") * 2 + lax.axis_index("c")
    base = wid * _BPW
    pltpu.sync_copy(idx_hbm.at[pl.ds(base, _BPW)], idx_v)
    obase = row0 + base

    bufs = (buf0, buf1, buf2)
    gsems = (gsem0, gsem1, gsem2)
    wsems = (wsem0, wsem1, wsem2)

    def start_gather(g):
        s = g % _NBUF
        return pltpu.async_copy(
            v_hbm.at[idx_v.at[pl.ds(g * _CHUNK, _CHUNK)]], bufs[s], gsems[s]
        )

    def start_write(g):
        s = g % _NBUF
        return pltpu.async_copy(
            bufs[s], out_hbm.at[pl.ds(obase + g * _CHUNK, _CHUNK)], wsems[s]
        )

    gh = [None] * _NBUF
    wh = [None] * _NBUF
    for g in range(_NCHUNK):
        s = g % _NBUF
        if wh[s] is not None:
            wh[s].wait()
        gh[s] = start_gather(g)
        if g >= 1:
            sp = (g - 1) % _NBUF
            gh[sp].wait()
            wh[sp] = start_write(g - 1)
    last = (_NCHUNK - 1) % _NBUF
    gh[last].wait()
    wh[last] = start_write(_NCHUNK - 1)
    for s in range(_NBUF):
        if wh[s] is not None:
            wh[s].wait()


_NBUF = 3

_SC_SCRATCH = (
    [pltpu.VMEM((_BPW,), jnp.int32)]
    + [pltpu.VMEM((_CHUNK, _DP), jnp.float32)] * _NBUF
    + [pltpu.SemaphoreType.DMA] * (2 * _NBUF)
)


@functools.cache
def _gather_half0():
    return functools.partial(
        pl.kernel,
        out_type=jax.ShapeDtypeStruct((_NB, _DP), jnp.float32),
        mesh=plsc.VectorSubcoreMesh(core_axis_name="c", subcore_axis_name="s"),
        scratch_types=list(_SC_SCRATCH),
    )(functools.partial(_gather_half_body, 0))


@functools.cache
def _gather_half1():
    def body(v_hbm, idx_hbm, out_ref, *scratch):
        _gather_half_body(_NBH, v_hbm, idx_hbm, out_ref, *scratch)

    return functools.partial(
        pl.kernel,
        out_type=(),
        mesh=plsc.VectorSubcoreMesh(core_axis_name="c", subcore_axis_name="s"),
        scratch_types=list(_SC_SCRATCH),
    )(body)


def kernel(q, k, v, Wq, Wk):
    v_flat = v.reshape(_B * _C, _DP)
    logits_h, idx0 = _half0_call(q, k, Wq, Wk)
    rec = _gather_half0()(v_flat, idx0.reshape(_NBH))
    logits, idx1 = _half1_call(q, k, Wq, Wk, logits_h)
    rec_ref = jax.new_ref(rec)
    _gather_half1()(v_flat, idx1.reshape(_NBH), rec_ref)
    rec_x = rec_ref[...].reshape(_B, _C, _TOPK, _DP)
    return (q, rec_x, logits)


# speedup vs baseline: 1.2666x; 1.0333x over previous
import functools

import jax
import jax.numpy as jnp
from jax import lax
from jax.experimental import pallas as pl
from jax.experimental.pallas import tpu as pltpu
from jax.experimental.pallas import tpu_sc as plsc

_B, _C, _DP = 4, 2048, 256
_DR = 64
_TOPK = 8
_TQ = 1024
_BH = _B // 2

_NB = _B * _C * _TOPK
_NBH = _NB // 2
_NW = 32
_BPW = _NBH // _NW
_CHUNK = 128
_NCHUNK = _BPW // _CHUNK


def _fused_body(b_base, q_ref, k_ref, wq_ref, wk_ref, logits_ref, idx_ref,
                xk_ref):
    b = pl.program_id(0)
    i = pl.program_id(1)

    @pl.when(i == 0)
    def _():
        xk_ref[...] = jnp.dot(
            k_ref[0], wk_ref[...], preferred_element_type=jnp.float32
        )

    x_q = jnp.dot(q_ref[0], wq_ref[...], preferred_element_type=jnp.float32)
    logits = lax.dot_general(
        x_q, xk_ref[...], (((1,), (1,)), ((), ())),
        preferred_element_type=jnp.float32,
    )
    logits_ref[0] = logits

    half = _C // 2
    colL = lax.broadcasted_iota(jnp.int32, (_TQ, half), 1).astype(jnp.float32)
    colR = colL + float(half)
    lhs = logits[:, :half]
    rhs = logits[:, half:]
    cond = lhs >= rhs
    P = jnp.maximum(lhs, rhs)
    NV = jnp.minimum(lhs, rhs)
    PC = jnp.where(cond, colL, colR)
    NC = jnp.where(cond, colR, colL)
    t_iota = lax.broadcasted_iota(jnp.int32, (_TQ, _TOPK), 1)
    idx_acc = jnp.zeros((_TQ, _TOPK), jnp.int32)
    for t in range(_TOPK):
        m = jnp.max(P, axis=1, keepdims=True)
        amaxc = jnp.min(jnp.where(P == m, PC, float(_C)), axis=1,
                        keepdims=True)
        winner = PC == amaxc
        idx_acc = jnp.where(t_iota == t, amaxc.astype(jnp.int32), idx_acc)
        P = jnp.where(winner, NV, P)
        PC = jnp.where(winner, NC, PC)
        NV = jnp.where(winner, -jnp.inf, NV)
    idx_ref[0] = idx_acc + (b + b_base) * _C


def _first_call(nb, q, k, Wq, Wk, interpret=False):
    return pl.pallas_call(
        functools.partial(_fused_body, 0),
        grid=(nb, _C // _TQ),
        in_specs=[
            pl.BlockSpec((1, _TQ, _DP), lambda b, i: (b, i, 0)),
            pl.BlockSpec((1, _C, _DP), lambda b, i: (b, 0, 0)),
            pl.BlockSpec((_DP, _DR), lambda b, i: (0, 0)),
            pl.BlockSpec((_DP, _DR), lambda b, i: (0, 0)),
        ],
        out_specs=[
            pl.BlockSpec((1, _TQ, _C), lambda b, i: (b, i, 0)),
            pl.BlockSpec((1, _TQ, _TOPK), lambda b, i: (b, i, 0)),
        ],
        out_shape=[
            jax.ShapeDtypeStruct((_B, _C, _C), jnp.float32),
            jax.ShapeDtypeStruct((nb, _C, _TOPK), jnp.int32),
        ],
        interpret=interpret,
        scratch_shapes=[pltpu.VMEM((_C, _DR), jnp.float32)],
    )(q, k, Wq, Wk)


def _next_body(b_base, q_ref, k_ref, wq_ref, wk_ref, prev_ref, logits_ref,
               idx_ref, xk_ref):
    del prev_ref
    _fused_body(b_base, q_ref, k_ref, wq_ref, wk_ref, logits_ref, idx_ref,
                xk_ref)


def _next_call(b_base, nb, q, k, Wq, Wk, logits_buf, interpret=False):
    return pl.pallas_call(
        functools.partial(_next_body, b_base),
        grid=(nb, _C // _TQ),
        in_specs=[
            pl.BlockSpec((1, _TQ, _DP), lambda b, i: (b + b_base, i, 0)),
            pl.BlockSpec((1, _C, _DP), lambda b, i: (b + b_base, 0, 0)),
            pl.BlockSpec((_DP, _DR), lambda b, i: (0, 0)),
            pl.BlockSpec((_DP, _DR), lambda b, i: (0, 0)),
            pl.BlockSpec(memory_space=pl.ANY),
        ],
        out_specs=[
            pl.BlockSpec((1, _TQ, _C), lambda b, i: (b + b_base, i, 0)),
            pl.BlockSpec((1, _TQ, _TOPK), lambda b, i: (b, i, 0)),
        ],
        out_shape=[
            jax.ShapeDtypeStruct((_B, _C, _C), jnp.float32),
            jax.ShapeDtypeStruct((nb, _C, _TOPK), jnp.int32),
        ],
        input_output_aliases={4: 0},
        interpret=interpret,
        scratch_shapes=[pltpu.VMEM((_C, _DR), jnp.float32)],
    )(q, k, Wq, Wk, logits_buf)


_NBUF = 3


def _gather_seg_body(row0, nrows, v_hbm, idx_hbm, out_hbm, idx_v,
                     buf0, buf1, buf2, gsem0, gsem1, gsem2,
                     wsem0, wsem1, wsem2):
    bpw = nrows // _NW
    nchunk = bpw // _CHUNK
    wid = lax.axis_index("s") * 2 + lax.axis_index("c")
    base = wid * bpw
    pltpu.sync_copy(idx_hbm.at[pl.ds(base, bpw)], idx_v)
    obase = row0 + base

    bufs = (buf0, buf1, buf2)
    gsems = (gsem0, gsem1, gsem2)
    wsems = (wsem0, wsem1, wsem2)

    def start_gather(g):
        s = g % _NBUF
        return pltpu.async_copy(
            v_hbm.at[idx_v.at[pl.ds(g * _CHUNK, _CHUNK)]], bufs[s], gsems[s]
        )

    def start_write(g):
        s = g % _NBUF
        return pltpu.async_copy(
            bufs[s], out_hbm.at[pl.ds(obase + g * _CHUNK, _CHUNK)], wsems[s]
        )

    gh = [None] * _NBUF
    wh = [None] * _NBUF
    for g in range(nchunk):
        s = g % _NBUF
        if wh[s] is not None:
            wh[s].wait()
        gh[s] = start_gather(g)
        if g >= 1:
            sp = (g - 1) % _NBUF
            gh[sp].wait()
            wh[sp] = start_write(g - 1)
    last = (nchunk - 1) % _NBUF
    gh[last].wait()
    wh[last] = start_write(nchunk - 1)
    for s in range(_NBUF):
        if wh[s] is not None:
            wh[s].wait()


@functools.cache
def _gather_seg(row0, nrows, first):
    scratch = (
        [pltpu.VMEM((nrows // _NW,), jnp.int32)]
        + [pltpu.VMEM((_CHUNK, _DP), jnp.float32)] * _NBUF
        + [pltpu.SemaphoreType.DMA] * (2 * _NBUF)
    )
    body = functools.partial(_gather_seg_body, row0, nrows)
    out_type = (
        jax.ShapeDtypeStruct((_NB, _DP), jnp.float32) if first else ()
    )
    return functools.partial(
        pl.kernel,
        out_type=out_type,
        mesh=plsc.VectorSubcoreMesh(core_axis_name="c", subcore_axis_name="s"),
        scratch_types=scratch,
    )(body)


def kernel(q, k, v, Wq, Wk):
    v_flat = v.reshape(_B * _C, _DP)
    nb0 = 2
    n0 = nb0 * _C * _TOPK
    n1 = _C * _TOPK
    logits_a, idx01 = _first_call(nb0, q, k, Wq, Wk)
    rec = _gather_seg(0, n0, True)(v_flat, idx01.reshape(n0))
    logits_b, idx2 = _next_call(2, 1, q, k, Wq, Wk, logits_a)
    rec_ref = jax.new_ref(rec)
    _gather_seg(n0, n1, False)(v_flat, idx2.reshape(n1), rec_ref)
    logits, idx3 = _next_call(3, 1, q, k, Wq, Wk, logits_b)
    _gather_seg(n0 + n1, n1, False)(v_flat, idx3.reshape(n1), rec_ref)
    rec_x = rec_ref[...].reshape(_B, _C, _TOPK, _DP)
    return (q, rec_x, logits)
